# baseline (speedup 1.0000x reference)
import functools

import jax
import jax.numpy as jnp
from jax import lax
from jax.experimental import pallas as pl
from jax.experimental.pallas import tpu as pltpu
from jax.experimental.pallas import tpu_sc as plsc

NUM_BINS = 6
STRIDE = 4
L = 16


def _alloc_points(bin_prob, max_num_points, stride):
    total = jnp.sum(max_num_points[0, :]) // stride
    B, num_bins = bin_prob.shape
    p = bin_prob * max_num_points.astype(bin_prob.dtype) + 1e-10
    chosen = jnp.zeros_like(p)
    mnp_f = max_num_points.astype(p.dtype)
    for _ in range(num_bins):
        p = p / jnp.sum(p, axis=1, keepdims=True)
        num_to_choose = total.astype(p.dtype) - jnp.sum(chosen, axis=1, keepdims=True)
        chosen = chosen + p * num_to_choose
        chosen = jnp.where(chosen >= mnp_f, mnp_f, chosen)
        p = p * jnp.where(chosen >= mnp_f, 0.0, 1.0)
    chosen = chosen.astype(jnp.int32)
    adj = jnp.argmax(mnp_f - chosen.astype(p.dtype), axis=1)
    deficit = total.astype(jnp.int32) - jnp.sum(chosen, axis=1)
    chosen = chosen.at[jnp.arange(B), adj].add(deficit)
    return chosen


def _asc_u32(x):
    ub = lax.bitcast_convert_type(x, jnp.uint32)
    return jnp.where(ub >= jnp.uint32(0x80000000), ~ub, ub | jnp.uint32(0x80000000))


def _asc_u32_inv(r):
    ub = jnp.where(r >= jnp.uint32(0x80000000), r ^ jnp.uint32(0x80000000), ~r)
    return lax.bitcast_convert_type(ub, jnp.float32)


def _order_stats(u_flat, pos):
    nt = pos.shape[0]
    i32 = jnp.int32

    b1 = (u_flat >> jnp.uint32(16)).astype(i32)
    hist1 = jnp.zeros((65536,), i32).at[b1].add(1)
    cum1 = jnp.cumsum(hist1)
    bkt = jnp.sum((cum1[None, :] <= pos[:, None]).astype(i32), axis=1)
    base1 = jnp.where(bkt > 0, jnp.take(cum1, jnp.maximum(bkt - 1, 0)), 0)
    pos2 = pos - base1

    sb = jnp.sort(bkt)
    sid_of = lambda v: jnp.sum((sb[None, :] < v[:, None]).astype(i32), axis=1)
    tsid = sid_of(bkt)

    byte2v = ((u_flat >> jnp.uint32(8)) & jnp.uint32(0xFF)).astype(i32)
    m2 = b1[:, None] == bkt[None, :]
    esid2 = jnp.sum((sb[None, :] < b1[:, None]).astype(i32), axis=1)
    sel2 = jnp.where(m2.any(axis=1), esid2 * 256 + byte2v, nt * 256)
    hist2 = jnp.zeros((nt * 256 + 1,), i32).at[sel2].add(1)
    cum2 = jnp.cumsum(hist2[:nt * 256].reshape(nt, 256), axis=1)
    mycum2 = jnp.take(cum2, tsid, axis=0)
    byte2 = jnp.sum((mycum2 <= pos2[:, None]).astype(i32), axis=1)
    base2 = jnp.where(byte2 > 0,
                      jnp.take_along_axis(mycum2, jnp.maximum(byte2 - 1, 0)[:, None],
                                          axis=1)[:, 0], 0)
    pos3 = pos2 - base2

    key_t = bkt * 256 + byte2
    sk = jnp.sort(key_t)
    tsid3 = jnp.sum((sk[None, :] < key_t[:, None]).astype(i32), axis=1)
    ekey = b1 * 256 + byte2v
    byte3v = (u_flat & jnp.uint32(0xFF)).astype(i32)
    m3 = ekey[:, None] == key_t[None, :]
    esid3 = jnp.sum((sk[None, :] < ekey[:, None]).astype(i32), axis=1)
    sel3 = jnp.where(m3.any(axis=1), esid3 * 256 + byte3v, nt * 256)
    hist3 = jnp.zeros((nt * 256 + 1,), i32).at[sel3].add(1)
    cum3 = jnp.cumsum(hist3[:nt * 256].reshape(nt, 256), axis=1)
    mycum3 = jnp.take(cum3, tsid3, axis=0)
    byte3 = jnp.sum((mycum3 <= pos3[:, None]).astype(i32), axis=1)

    r = ((bkt.astype(jnp.uint32) << 16)
         | (byte2.astype(jnp.uint32) << 8) | byte3.astype(jnp.uint32))
    return r


def _make_sc_kernel(B, N, D, M):
    CH = N // L
    mesh = plsc.VectorSubcoreMesh(core_axis_name="c", subcore_axis_name="s")
    NC = 2

    @functools.partial(
        pl.kernel,
        out_type=(
            jax.ShapeDtypeStruct((B, M, D), jnp.float32),
            jax.ShapeDtypeStruct((B, M), jnp.int32),
        ),
        mesh=mesh,
        scratch_types=[
            pltpu.VMEM((N,), jnp.int32),
            pltpu.VMEM((N,), jnp.int32),
            pltpu.VMEM((N,), jnp.int32),
            pltpu.VMEM((N,), jnp.int32),
            pltpu.VMEM((N,), jnp.int32),
            pltpu.VMEM((8 * L,), jnp.int32),
            pltpu.VMEM((M,), jnp.int32),
            pltpu.VMEM((M,), jnp.int32),
            pltpu.VMEM((M,), jnp.int32),
            pltpu.VMEM((M // 8, 2 * D), jnp.float32),
            pltpu.VMEM((M // 8, D), jnp.float32),
            pltpu.VMEM((L,), jnp.int32),
            pltpu.VMEM((L,), jnp.int32),
            pltpu.SemaphoreType.DMA,
        ],
        compiler_params=pltpu.CompilerParams(needs_layout_passes=False),
    )
    def grouper(key_h, val_h, prm_h, pts_h, npo_h, idxo_h,
                keyA, valA, keyB, valB, chunk, prmv, outidx, idxadj, parbuf,
                gbuf, stage, c1, c3, sem):
        cid = lax.axis_index("c")
        sid = lax.axis_index("s")
        b = sid * NC + cid

        pltpu.sync_copy(key_h.at[b], keyA)
        pltpu.sync_copy(val_h.at[b], valA)
        pltpu.sync_copy(prm_h.at[b], prmv)

        zvec = jnp.zeros((L,), jnp.int32)

        bufs = [(keyA, valA, keyB, valB), (keyB, valB, keyA, valA)]
        for p in range(8):
            Kin, Vin, Kout, Vout = bufs[p % 2]
            sh = jnp.int32(4 * p)

            @plsc.parallel_loop(0, CH, unroll=4)
            def hist_body(c, Kin=Kin, sh=sh):
                kv = Kin[pl.ds(c * L, L)]
                d = lax.shift_right_logical(kv, sh) & jnp.int32(15)
                cnt, last = plsc.scan_count(d)
                chunk[pl.ds(c * L, L)] = zvec
                plsc.store_scatter(chunk, [c * L + d], cnt, mask=last)

            def prefix_body(c, acc):
                row = chunk[pl.ds(c * L, L)]
                chunk[pl.ds(c * L, L)] = acc
                return acc + row

            tot = lax.fori_loop(0, CH, prefix_body, zvec)
            gbase = plsc.cumsum(tot) - tot

            @plsc.parallel_loop(0, CH, unroll=4)
            def perm_body(c, Kin=Kin, Vin=Vin, Kout=Kout, Vout=Vout,
                          sh=sh, gbase=gbase):
                kv = Kin[pl.ds(c * L, L)]
                vv = Vin[pl.ds(c * L, L)]
                d = lax.shift_right_logical(kv, sh) & jnp.int32(15)
                cnt, _last = plsc.scan_count(d)
                basev = gbase + chunk[pl.ds(c * L, L)]
                rank = jnp.take(basev, d, axis=0) + cnt - 1
                plsc.store_scatter(Kout, [rank], kv)
                plsc.store_scatter(Vout, [rank], vv)

        startv = prmv[pl.ds(0 * L, L)]
        a1v = prmv[pl.ds(1 * L, L)]
        a2v = prmv[pl.ds(2 * L, L)]
        a3v = prmv[pl.ds(3 * L, L)]
        s2v = prmv[pl.ds(4 * L, L)]
        s3v = prmv[pl.ds(5 * L, L)]
        c1[...] = zvec
        c3[...] = zvec

        def selA(c, _):
            vv = valA[pl.ds(c * L, L)]
            gv = (vv >> 16) & jnp.int32(7)
            cls = (vv >> 20) & jnp.int32(3)
            iv = vv & jnp.int32(0xFFFF)
            m1 = cls == 0
            cnt1, last1 = plsc.scan_count(gv, m1)
            r1 = plsc.load_gather(c1, [gv]) + cnt1 - 1
            sel1 = m1 & (r1 < jnp.take(a1v, gv, axis=0))
            plsc.store_scatter(
                outidx, [jnp.take(startv, gv, axis=0) + r1], iv, mask=sel1)
            plsc.addupdate_scatter(c1, [gv], cnt1, mask=last1)
            m3 = cls == 2
            cnt3, last3 = plsc.scan_count(gv, m3)
            r3 = plsc.load_gather(c3, [gv]) + cnt3 - 1
            sel3 = m3 & (r3 < jnp.take(a3v, gv, axis=0))
            plsc.store_scatter(
                outidx, [jnp.take(s3v, gv, axis=0) + r3], iv, mask=sel3)
            plsc.addupdate_scatter(c3, [gv], cnt3, mask=last3)
            return 0

        lax.fori_loop(0, N // L, selA, 0)

        pltpu.sync_copy(val_h.at[b], valB)
        a2b = [jnp.take(a2v, jnp.full((L,), j, jnp.int32), axis=0)
               for j in range(NUM_BINS)]
        s2b = [jnp.take(s2v, jnp.full((L,), j, jnp.int32), axis=0)
               for j in range(NUM_BINS)]
        lane15 = jnp.full((L,), L - 1, jnp.int32)

        def selB(c, bases):
            vv = valB[pl.ds(c * L, L)]
            gv = (vv >> 16) & jnp.int32(7)
            cls = (vv >> 20) & jnp.int32(3)
            iv = vv & jnp.int32(0xFFFF)
            new_bases = []
            for j in range(NUM_BINS):
                elig = (gv != j) | (cls == 1)
                ei = jnp.where(elig, 1, 0).astype(jnp.int32)
                incl = plsc.cumsum(ei)
                excl = bases[j] + incl - ei
                sel = elig & (excl < a2b[j])
                plsc.store_scatter(outidx, [s2b[j] + excl], iv, mask=sel)
                new_bases.append(bases[j] + jnp.take(incl, lane15, axis=0))
            return tuple(new_bases)

        lax.fori_loop(0, N // L, selB, (zvec,) * NUM_BINS)

        pltpu.sync_copy(outidx, idxo_h.at[b])

        off = b * (N // 2)

        @plsc.parallel_loop(0, M // L, unroll=4)
        def adj_body(c):
            ov = outidx[pl.ds(c * L, L)]
            idxadj[pl.ds(c * L, L)] = (
                lax.shift_right_logical(ov, jnp.int32(1)) + off)
            parbuf[pl.ds(c * L, L)] = ov & jnp.int32(1)

        iota = lax.broadcasted_iota(jnp.int32, (L,), 0)
        q = M // 8
        for chk in range(8):
            cp = pltpu.async_copy(
                pts_h.at[idxadj.at[pl.ds(chk * q, q)]], gbuf, sem)
            cp.wait()

            @plsc.parallel_loop(0, 4 * q, unroll=4)
            def fill_body(i, chk=chk):
                r = lax.shift_right_logical(i, jnp.int32(2))
                qp = i & jnp.int32(3)
                rv = jnp.full((L,), r, jnp.int32)
                par = plsc.load_gather(parbuf, [chk * q + rv])
                dcol = qp * L + iota
                vals = plsc.load_gather(gbuf, [rv, par * D + dcol])
                plsc.store_scatter(stage, [rv, dcol], vals)

            pltpu.sync_copy(stage, npo_h.at[b, pl.ds(chk * q, q)])

    return grouper


def kernel(points, attention_point_score, bin_prob_logits):
    B, N, D = points.shape
    M = N // STRIDE
    score = attention_point_score

    m = jnp.mean(score, axis=2, keepdims=True)
    sd = jnp.std(score, axis=2, keepdims=True)
    s = ((score - m) / sd)[:, 0, :]

    n = B * N
    idxq = (jnp.arange(1, NUM_BINS) / NUM_BINS * n).astype(jnp.int32)
    pos_asc = jnp.int32(n - 1) - idxq
    bvals = _asc_u32_inv(_order_stats(_asc_u32(s.reshape(-1)), pos_asc))

    g = jnp.sum((s[:, :, None] < bvals[None, None, :]).astype(jnp.int32),
                axis=2)
    c_thr = jnp.float32(-1e-8)
    ge = jnp.sum((s[:, :, None] >= bvals[None, None, :]).astype(jnp.int32),
                 axis=1)
    gp = jnp.sum(((s[:, :, None] >= bvals[None, None, :])
                  & (s[:, :, None] > c_thr)).astype(jnp.int32), axis=1)
    gn = jnp.sum(((s[:, :, None] >= bvals[None, None, :])
                  & (s[:, :, None] < c_thr)).astype(jnp.int32), axis=1)
    npos = jnp.sum((s > c_thr).astype(jnp.int32), axis=1, keepdims=True)
    nneg = jnp.sum((s < c_thr).astype(jnp.int32), axis=1, keepdims=True)
    zero_col = jnp.zeros((B, 1), jnp.int32)
    ge_full = jnp.concatenate([zero_col, ge, jnp.full((B, 1), N, jnp.int32)], axis=1)
    gp_full = jnp.concatenate([zero_col, gp, npos], axis=1)
    gn_full = jnp.concatenate([zero_col, gn, nneg], axis=1)
    counts = ge_full[:, 1:] - ge_full[:, :-1]

    bin_prob = jnp.broadcast_to(
        jax.nn.softmax(bin_prob_logits)[None, :], (B, NUM_BINS))
    k = _alloc_points(bin_prob, counts, STRIDE)
    start = jnp.concatenate(
        [jnp.zeros((B, 1), jnp.int32), jnp.cumsum(k, axis=1)[:, :-1]], axis=1)

    t = s + jnp.float32(1e-8)
    key = lax.bitcast_convert_type(~_asc_u32(t), jnp.int32)
    cls = jnp.where(t > 0, 0, jnp.where(t < 0, 2, 1)).astype(jnp.int32)
    val = (jnp.broadcast_to(jnp.arange(N, dtype=jnp.int32)[None, :], (B, N))
           | (g << 16) | (cls << 20))

    n1 = gp_full[:, 1:] - gp_full[:, :-1]
    n4 = gn_full[:, 1:] - gn_full[:, :-1]
    a1 = jnp.minimum(k, n1)
    a2 = jnp.minimum(k - a1, N - n1 - n4)
    a3 = k - a1 - a2

    def pad16(x):
        return jnp.pad(x, ((0, 0), (0, L - NUM_BINS)))

    prm = jnp.stack(
        [pad16(start), pad16(a1), pad16(a2), pad16(a3),
         pad16(start + a1), pad16(start + a1 + a2),
         jnp.zeros((B, L), jnp.int32), jnp.zeros((B, L), jnp.int32)],
        axis=1).reshape(B, 8 * L)

    grouper = _make_sc_kernel(B, N, D, M)
    new_points, idx = grouper(key, val, prm, points.reshape(B * N // 2, 2 * D))
    return new_points, idx.reshape(B, 1, M)


# speedup vs baseline: 1.9758x; 1.9758x over previous
import functools

import jax
import jax.numpy as jnp
from jax import lax
from jax.experimental import pallas as pl
from jax.experimental.pallas import tpu as pltpu
from jax.experimental.pallas import tpu_sc as plsc

NUM_BINS = 6
STRIDE = 4
L = 16


def _alloc_points(bin_prob, max_num_points, stride):
    total = jnp.sum(max_num_points[0, :]) // stride
    B, num_bins = bin_prob.shape
    p = bin_prob * max_num_points.astype(bin_prob.dtype) + 1e-10
    chosen = jnp.zeros_like(p)
    mnp_f = max_num_points.astype(p.dtype)
    for _ in range(num_bins):
        p = p / jnp.sum(p, axis=1, keepdims=True)
        num_to_choose = total.astype(p.dtype) - jnp.sum(chosen, axis=1, keepdims=True)
        chosen = chosen + p * num_to_choose
        chosen = jnp.where(chosen >= mnp_f, mnp_f, chosen)
        p = p * jnp.where(chosen >= mnp_f, 0.0, 1.0)
    chosen = chosen.astype(jnp.int32)
    adj = jnp.argmax(mnp_f - chosen.astype(p.dtype), axis=1)
    deficit = total.astype(jnp.int32) - jnp.sum(chosen, axis=1)
    chosen = chosen.at[jnp.arange(B), adj].add(deficit)
    return chosen


def _asc_u32(x):
    ub = lax.bitcast_convert_type(x, jnp.uint32)
    return jnp.where(ub >= jnp.uint32(0x80000000), ~ub, ub | jnp.uint32(0x80000000))


def _asc_u32_inv(r):
    ub = jnp.where(r >= jnp.uint32(0x80000000), r ^ jnp.uint32(0x80000000), ~r)
    return lax.bitcast_convert_type(ub, jnp.float32)


def _order_stats(u_flat, pos):
    nt = pos.shape[0]

    def body(i, r):
        sh = (jnp.uint32(30) - 2 * i.astype(jnp.uint32))
        cands = (r[:, None]
                 | (jnp.uint32(1) + jnp.arange(3, dtype=jnp.uint32))[None, :] << sh)
        cnt = jnp.sum((cands.reshape(-1)[:, None] > u_flat[None, :]).astype(jnp.int32),
                      axis=1).reshape(nt, 3)
        d = jnp.sum((cnt <= pos[:, None]).astype(jnp.uint32), axis=1)
        return r | (d << sh)

    return lax.fori_loop(0, 16, body, jnp.zeros(pos.shape, jnp.uint32))


def _make_sc_kernel(B, N, D, M):
    CH = N // L
    mesh = plsc.VectorSubcoreMesh(core_axis_name="c", subcore_axis_name="s")
    NC = 2

    @functools.partial(
        pl.kernel,
        out_type=(
            jax.ShapeDtypeStruct((B, M, D), jnp.float32),
            jax.ShapeDtypeStruct((B, M), jnp.int32),
        ),
        mesh=mesh,
        scratch_types=[
            pltpu.VMEM((N,), jnp.int32),
            pltpu.VMEM((N,), jnp.int32),
            pltpu.VMEM((N,), jnp.int32),
            pltpu.VMEM((N,), jnp.int32),
            pltpu.VMEM((N,), jnp.int32),
            pltpu.VMEM((8 * L,), jnp.int32),
            pltpu.VMEM((M,), jnp.int32),
            pltpu.VMEM((M,), jnp.int32),
            pltpu.VMEM((M,), jnp.int32),
            pltpu.VMEM((M // 8, 2 * D), jnp.float32),
            pltpu.VMEM((M // 8, D), jnp.float32),
            pltpu.VMEM((L,), jnp.int32),
            pltpu.VMEM((L,), jnp.int32),
            pltpu.SemaphoreType.DMA,
        ],
        compiler_params=pltpu.CompilerParams(needs_layout_passes=False),
    )
    def grouper(key_h, val_h, prm_h, pts_h, npo_h, idxo_h,
                keyA, valA, keyB, valB, chunk, prmv, outidx, idxadj, parbuf,
                gbuf, stage, c1, c3, sem):
        cid = lax.axis_index("c")
        sid = lax.axis_index("s")
        b = sid * NC + cid

        pltpu.sync_copy(key_h.at[b], keyA)
        pltpu.sync_copy(val_h.at[b], valA)
        pltpu.sync_copy(prm_h.at[b], prmv)

        zvec = jnp.zeros((L,), jnp.int32)

        bufs = [(keyA, valA, keyB, valB), (keyB, valB, keyA, valA)]
        for p in range(8):
            Kin, Vin, Kout, Vout = bufs[p % 2]
            sh = jnp.int32(4 * p)

            @plsc.parallel_loop(0, CH, unroll=4)
            def hist_body(c, Kin=Kin, sh=sh):
                kv = Kin[pl.ds(c * L, L)]
                d = lax.shift_right_logical(kv, sh) & jnp.int32(15)
                cnt, last = plsc.scan_count(d)
                chunk[pl.ds(c * L, L)] = zvec
                plsc.store_scatter(chunk, [c * L + d], cnt, mask=last)

            def prefix_body(c, acc):
                row = chunk[pl.ds(c * L, L)]
                chunk[pl.ds(c * L, L)] = acc
                return acc + row

            tot = lax.fori_loop(0, CH, prefix_body, zvec)
            gbase = plsc.cumsum(tot) - tot

            @plsc.parallel_loop(0, CH, unroll=4)
            def perm_body(c, Kin=Kin, Vin=Vin, Kout=Kout, Vout=Vout,
                          sh=sh, gbase=gbase):
                kv = Kin[pl.ds(c * L, L)]
                vv = Vin[pl.ds(c * L, L)]
                d = lax.shift_right_logical(kv, sh) & jnp.int32(15)
                cnt, _last = plsc.scan_count(d)
                basev = gbase + chunk[pl.ds(c * L, L)]
                rank = jnp.take(basev, d, axis=0) + cnt - 1
                plsc.store_scatter(Kout, [rank], kv)
                plsc.store_scatter(Vout, [rank], vv)

        startv = prmv[pl.ds(0 * L, L)]
        a1v = prmv[pl.ds(1 * L, L)]
        a2v = prmv[pl.ds(2 * L, L)]
        a3v = prmv[pl.ds(3 * L, L)]
        s2v = prmv[pl.ds(4 * L, L)]
        s3v = prmv[pl.ds(5 * L, L)]
        c1[...] = zvec
        c3[...] = zvec

        def selA(c, _):
            vv = valA[pl.ds(c * L, L)]
            gv = (vv >> 16) & jnp.int32(7)
            cls = (vv >> 20) & jnp.int32(3)
            iv = vv & jnp.int32(0xFFFF)
            m1 = cls == 0
            cnt1, last1 = plsc.scan_count(gv, m1)
            r1 = plsc.load_gather(c1, [gv]) + cnt1 - 1
            sel1 = m1 & (r1 < jnp.take(a1v, gv, axis=0))
            plsc.store_scatter(
                outidx, [jnp.take(startv, gv, axis=0) + r1], iv, mask=sel1)
            plsc.addupdate_scatter(c1, [gv], cnt1, mask=last1)
            m3 = cls == 2
            cnt3, last3 = plsc.scan_count(gv, m3)
            r3 = plsc.load_gather(c3, [gv]) + cnt3 - 1
            sel3 = m3 & (r3 < jnp.take(a3v, gv, axis=0))
            plsc.store_scatter(
                outidx, [jnp.take(s3v, gv, axis=0) + r3], iv, mask=sel3)
            plsc.addupdate_scatter(c3, [gv], cnt3, mask=last3)
            return 0

        lax.fori_loop(0, N // L, selA, 0)

        pltpu.sync_copy(val_h.at[b], valB)
        a2b = [jnp.take(a2v, jnp.full((L,), j, jnp.int32), axis=0)
               for j in range(NUM_BINS)]
        s2b = [jnp.take(s2v, jnp.full((L,), j, jnp.int32), axis=0)
               for j in range(NUM_BINS)]
        lane15 = jnp.full((L,), L - 1, jnp.int32)

        def selB(c, bases):
            vv = valB[pl.ds(c * L, L)]
            gv = (vv >> 16) & jnp.int32(7)
            cls = (vv >> 20) & jnp.int32(3)
            iv = vv & jnp.int32(0xFFFF)
            new_bases = []
            for j in range(NUM_BINS):
                elig = (gv != j) | (cls == 1)
                ei = jnp.where(elig, 1, 0).astype(jnp.int32)
                incl = plsc.cumsum(ei)
                excl = bases[j] + incl - ei
                sel = elig & (excl < a2b[j])
                plsc.store_scatter(outidx, [s2b[j] + excl], iv, mask=sel)
                new_bases.append(bases[j] + jnp.take(incl, lane15, axis=0))
            return tuple(new_bases)

        lax.fori_loop(0, N // L, selB, (zvec,) * NUM_BINS)

        pltpu.sync_copy(outidx, idxo_h.at[b])

        off = b * (N // 2)

        @plsc.parallel_loop(0, M // L, unroll=4)
        def adj_body(c):
            ov = outidx[pl.ds(c * L, L)]
            idxadj[pl.ds(c * L, L)] = (
                lax.shift_right_logical(ov, jnp.int32(1)) + off)
            parbuf[pl.ds(c * L, L)] = ov & jnp.int32(1)

        iota = lax.broadcasted_iota(jnp.int32, (L,), 0)
        q = M // 8
        for chk in range(8):
            cp = pltpu.async_copy(
                pts_h.at[idxadj.at[pl.ds(chk * q, q)]], gbuf, sem)
            cp.wait()

            @plsc.parallel_loop(0, 4 * q, unroll=4)
            def fill_body(i, chk=chk):
                r = lax.shift_right_logical(i, jnp.int32(2))
                qp = i & jnp.int32(3)
                rv = jnp.full((L,), r, jnp.int32)
                par = plsc.load_gather(parbuf, [chk * q + rv])
                dcol = qp * L + iota
                vals = plsc.load_gather(gbuf, [rv, par * D + dcol])
                plsc.store_scatter(stage, [rv, dcol], vals)

            pltpu.sync_copy(stage, npo_h.at[b, pl.ds(chk * q, q)])

    return grouper


def kernel(points, attention_point_score, bin_prob_logits):
    B, N, D = points.shape
    M = N // STRIDE
    score = attention_point_score

    m = jnp.mean(score, axis=2, keepdims=True)
    sd = jnp.std(score, axis=2, keepdims=True)
    s = ((score - m) / sd)[:, 0, :]

    n = B * N
    idxq = (jnp.arange(1, NUM_BINS) / NUM_BINS * n).astype(jnp.int32)
    pos_asc = jnp.int32(n - 1) - idxq
    bvals = _asc_u32_inv(_order_stats(_asc_u32(s.reshape(-1)), pos_asc))

    g = jnp.sum((s[:, :, None] < bvals[None, None, :]).astype(jnp.int32),
                axis=2)
    c_thr = jnp.float32(-1e-8)
    ge = jnp.sum((s[:, :, None] >= bvals[None, None, :]).astype(jnp.int32),
                 axis=1)
    gp = jnp.sum(((s[:, :, None] >= bvals[None, None, :])
                  & (s[:, :, None] > c_thr)).astype(jnp.int32), axis=1)
    gn = jnp.sum(((s[:, :, None] >= bvals[None, None, :])
                  & (s[:, :, None] < c_thr)).astype(jnp.int32), axis=1)
    npos = jnp.sum((s > c_thr).astype(jnp.int32), axis=1, keepdims=True)
    nneg = jnp.sum((s < c_thr).astype(jnp.int32), axis=1, keepdims=True)
    zero_col = jnp.zeros((B, 1), jnp.int32)
    ge_full = jnp.concatenate([zero_col, ge, jnp.full((B, 1), N, jnp.int32)], axis=1)
    gp_full = jnp.concatenate([zero_col, gp, npos], axis=1)
    gn_full = jnp.concatenate([zero_col, gn, nneg], axis=1)
    counts = ge_full[:, 1:] - ge_full[:, :-1]

    bin_prob = jnp.broadcast_to(
        jax.nn.softmax(bin_prob_logits)[None, :], (B, NUM_BINS))
    k = _alloc_points(bin_prob, counts, STRIDE)
    start = jnp.concatenate(
        [jnp.zeros((B, 1), jnp.int32), jnp.cumsum(k, axis=1)[:, :-1]], axis=1)

    t = s + jnp.float32(1e-8)
    key = lax.bitcast_convert_type(~_asc_u32(t), jnp.int32)
    cls = jnp.where(t > 0, 0, jnp.where(t < 0, 2, 1)).astype(jnp.int32)
    val = (jnp.broadcast_to(jnp.arange(N, dtype=jnp.int32)[None, :], (B, N))
           | (g << 16) | (cls << 20))

    n1 = gp_full[:, 1:] - gp_full[:, :-1]
    n4 = gn_full[:, 1:] - gn_full[:, :-1]
    a1 = jnp.minimum(k, n1)
    a2 = jnp.minimum(k - a1, N - n1 - n4)
    a3 = k - a1 - a2

    def pad16(x):
        return jnp.pad(x, ((0, 0), (0, L - NUM_BINS)))

    prm = jnp.stack(
        [pad16(start), pad16(a1), pad16(a2), pad16(a3),
         pad16(start + a1), pad16(start + a1 + a2),
         jnp.zeros((B, L), jnp.int32), jnp.zeros((B, L), jnp.int32)],
        axis=1).reshape(B, 8 * L)

    grouper = _make_sc_kernel(B, N, D, M)
    new_points, idx = grouper(key, val, prm, points.reshape(B * N // 2, 2 * D))
    return new_points, idx.reshape(B, 1, M)


# speedup vs baseline: 1.9831x; 1.0037x over previous
import functools

import jax
import jax.numpy as jnp
from jax import lax
from jax.experimental import pallas as pl
from jax.experimental.pallas import tpu as pltpu
from jax.experimental.pallas import tpu_sc as plsc

NUM_BINS = 6
STRIDE = 4
L = 16


def _alloc_points(bin_prob, max_num_points, stride):
    total = jnp.sum(max_num_points[0, :]) // stride
    B, num_bins = bin_prob.shape
    p = bin_prob * max_num_points.astype(bin_prob.dtype) + 1e-10
    chosen = jnp.zeros_like(p)
    mnp_f = max_num_points.astype(p.dtype)
    for _ in range(num_bins):
        p = p / jnp.sum(p, axis=1, keepdims=True)
        num_to_choose = total.astype(p.dtype) - jnp.sum(chosen, axis=1, keepdims=True)
        chosen = chosen + p * num_to_choose
        chosen = jnp.where(chosen >= mnp_f, mnp_f, chosen)
        p = p * jnp.where(chosen >= mnp_f, 0.0, 1.0)
    chosen = chosen.astype(jnp.int32)
    adj = jnp.argmax(mnp_f - chosen.astype(p.dtype), axis=1)
    deficit = total.astype(jnp.int32) - jnp.sum(chosen, axis=1)
    chosen = chosen.at[jnp.arange(B), adj].add(deficit)
    return chosen


def _asc_u32(x):
    ub = lax.bitcast_convert_type(x, jnp.uint32)
    return jnp.where(ub >= jnp.uint32(0x80000000), ~ub, ub | jnp.uint32(0x80000000))


def _asc_u32_inv(r):
    ub = jnp.where(r >= jnp.uint32(0x80000000), r ^ jnp.uint32(0x80000000), ~r)
    return lax.bitcast_convert_type(ub, jnp.float32)


def _order_stats(u_flat, pos):
    nt = pos.shape[0]

    def body(i, r):
        sh = (jnp.uint32(30) - 2 * i.astype(jnp.uint32))
        cands = (r[:, None]
                 | (jnp.uint32(1) + jnp.arange(3, dtype=jnp.uint32))[None, :] << sh)
        cnt = jnp.sum((cands.reshape(-1)[:, None] > u_flat[None, :]).astype(jnp.int32),
                      axis=1).reshape(nt, 3)
        d = jnp.sum((cnt <= pos[:, None]).astype(jnp.uint32), axis=1)
        return r | (d << sh)

    return lax.fori_loop(0, 16, body, jnp.zeros(pos.shape, jnp.uint32))


def _make_sc_kernel(B, N, D, M):
    CH = N // L
    mesh = plsc.VectorSubcoreMesh(core_axis_name="c", subcore_axis_name="s")
    NC = 2

    @functools.partial(
        pl.kernel,
        out_type=(
            jax.ShapeDtypeStruct((B * M, D), jnp.float32),
            jax.ShapeDtypeStruct((B * M,), jnp.int32),
        ),
        mesh=mesh,
        scratch_types=[
            pltpu.VMEM((N,), jnp.int32),
            pltpu.VMEM((N,), jnp.int32),
            pltpu.VMEM((N,), jnp.int32),
            pltpu.VMEM((N,), jnp.int32),
            pltpu.VMEM((N,), jnp.int32),
            pltpu.VMEM((8 * L,), jnp.int32),
            pltpu.VMEM((M,), jnp.int32),
            pltpu.VMEM((M,), jnp.int32),
            pltpu.VMEM((M,), jnp.int32),
            pltpu.VMEM((M // 8, 2 * D), jnp.float32),
            pltpu.VMEM((M // 8, D), jnp.float32),
            pltpu.VMEM((L,), jnp.int32),
            pltpu.VMEM((L,), jnp.int32),
            pltpu.SemaphoreType.DMA,
        ],
        compiler_params=pltpu.CompilerParams(needs_layout_passes=False),
    )
    def grouper(key_h, val_h, prm_h, pts_h, npo_h, idxo_h,
                keyA, valA, keyB, valB, chunk, prmv, outidx, idxadj, parbuf,
                gbuf, stage, c1, c3, sem):
        cid = lax.axis_index("c")
        sid = lax.axis_index("s")
        b = sid * NC + cid

        pltpu.sync_copy(key_h.at[pl.ds(b * N, N)], keyA)
        pltpu.sync_copy(val_h.at[pl.ds(b * N, N)], valA)
        pltpu.sync_copy(prm_h.at[pl.ds(b * 8 * L, 8 * L)], prmv)

        zvec = jnp.zeros((L,), jnp.int32)

        bufs = [(keyA, valA, keyB, valB), (keyB, valB, keyA, valA)]
        for p in range(8):
            Kin, Vin, Kout, Vout = bufs[p % 2]
            sh = jnp.int32(4 * p)

            @plsc.parallel_loop(0, CH, unroll=4)
            def hist_body(c, Kin=Kin, sh=sh):
                kv = Kin[pl.ds(c * L, L)]
                d = lax.shift_right_logical(kv, sh) & jnp.int32(15)
                cnt, last = plsc.scan_count(d)
                chunk[pl.ds(c * L, L)] = zvec
                plsc.store_scatter(chunk, [c * L + d], cnt, mask=last)

            def prefix_body(c, acc):
                row = chunk[pl.ds(c * L, L)]
                chunk[pl.ds(c * L, L)] = acc
                return acc + row

            tot = lax.fori_loop(0, CH, prefix_body, zvec)
            gbase = plsc.cumsum(tot) - tot

            @plsc.parallel_loop(0, CH, unroll=4)
            def perm_body(c, Kin=Kin, Vin=Vin, Kout=Kout, Vout=Vout,
                          sh=sh, gbase=gbase):
                kv = Kin[pl.ds(c * L, L)]
                vv = Vin[pl.ds(c * L, L)]
                d = lax.shift_right_logical(kv, sh) & jnp.int32(15)
                cnt, _last = plsc.scan_count(d)
                basev = gbase + chunk[pl.ds(c * L, L)]
                rank = jnp.take(basev, d, axis=0) + cnt - 1
                plsc.store_scatter(Kout, [rank], kv)
                plsc.store_scatter(Vout, [rank], vv)

        startv = prmv[pl.ds(0 * L, L)]
        a1v = prmv[pl.ds(1 * L, L)]
        a2v = prmv[pl.ds(2 * L, L)]
        a3v = prmv[pl.ds(3 * L, L)]
        s2v = prmv[pl.ds(4 * L, L)]
        s3v = prmv[pl.ds(5 * L, L)]
        c1[...] = zvec
        c3[...] = zvec

        def selA(c, _):
            vv = valA[pl.ds(c * L, L)]
            gv = (vv >> 16) & jnp.int32(7)
            cls = (vv >> 20) & jnp.int32(3)
            iv = vv & jnp.int32(0xFFFF)
            m1 = cls == 0
            cnt1, last1 = plsc.scan_count(gv, m1)
            r1 = plsc.load_gather(c1, [gv]) + cnt1 - 1
            sel1 = m1 & (r1 < jnp.take(a1v, gv, axis=0))
            plsc.store_scatter(
                outidx, [jnp.take(startv, gv, axis=0) + r1], iv, mask=sel1)
            plsc.addupdate_scatter(c1, [gv], cnt1, mask=last1)
            m3 = cls == 2
            cnt3, last3 = plsc.scan_count(gv, m3)
            r3 = plsc.load_gather(c3, [gv]) + cnt3 - 1
            sel3 = m3 & (r3 < jnp.take(a3v, gv, axis=0))
            plsc.store_scatter(
                outidx, [jnp.take(s3v, gv, axis=0) + r3], iv, mask=sel3)
            plsc.addupdate_scatter(c3, [gv], cnt3, mask=last3)
            return 0

        lax.fori_loop(0, N // L, selA, 0)

        pltpu.sync_copy(val_h.at[pl.ds(b * N, N)], valB)
        a2b = [jnp.take(a2v, jnp.full((L,), j, jnp.int32), axis=0)
               for j in range(NUM_BINS)]
        s2b = [jnp.take(s2v, jnp.full((L,), j, jnp.int32), axis=0)
               for j in range(NUM_BINS)]
        lane15 = jnp.full((L,), L - 1, jnp.int32)

        def selB(c, bases):
            vv = valB[pl.ds(c * L, L)]
            gv = (vv >> 16) & jnp.int32(7)
            cls = (vv >> 20) & jnp.int32(3)
            iv = vv & jnp.int32(0xFFFF)
            new_bases = []
            for j in range(NUM_BINS):
                elig = (gv != j) | (cls == 1)
                ei = jnp.where(elig, 1, 0).astype(jnp.int32)
                incl = plsc.cumsum(ei)
                excl = bases[j] + incl - ei
                sel = elig & (excl < a2b[j])
                plsc.store_scatter(outidx, [s2b[j] + excl], iv, mask=sel)
                new_bases.append(bases[j] + jnp.take(incl, lane15, axis=0))
            return tuple(new_bases)

        lax.fori_loop(0, N // L, selB, (zvec,) * NUM_BINS)

        pltpu.sync_copy(outidx, idxo_h.at[pl.ds(b * M, M)])

        off = b * (N // 2)

        @plsc.parallel_loop(0, M // L, unroll=4)
        def adj_body(c):
            ov = outidx[pl.ds(c * L, L)]
            idxadj[pl.ds(c * L, L)] = (
                lax.shift_right_logical(ov, jnp.int32(1)) + off)
            parbuf[pl.ds(c * L, L)] = ov & jnp.int32(1)

        iota = lax.broadcasted_iota(jnp.int32, (L,), 0)
        q = M // 8
        for chk in range(8):
            cp = pltpu.async_copy(
                pts_h.at[idxadj.at[pl.ds(chk * q, q)]], gbuf, sem)
            cp.wait()

            @plsc.parallel_loop(0, 4 * q, unroll=4)
            def fill_body(i, chk=chk):
                r = lax.shift_right_logical(i, jnp.int32(2))
                qp = i & jnp.int32(3)
                rv = jnp.full((L,), r, jnp.int32)
                par = plsc.load_gather(parbuf, [chk * q + rv])
                dcol = qp * L + iota
                vals = plsc.load_gather(gbuf, [rv, par * D + dcol])
                plsc.store_scatter(stage, [rv, dcol], vals)

            pltpu.sync_copy(stage, npo_h.at[pl.ds(b * M + chk * q, q)])

    return grouper


def kernel(points, attention_point_score, bin_prob_logits):
    B, N, D = points.shape
    M = N // STRIDE
    score = attention_point_score

    m = jnp.mean(score, axis=2, keepdims=True)
    sd = jnp.std(score, axis=2, keepdims=True)
    s = ((score - m) / sd)[:, 0, :]

    n = B * N
    idxq = (jnp.arange(1, NUM_BINS) / NUM_BINS * n).astype(jnp.int32)
    pos_asc = jnp.int32(n - 1) - idxq
    bvals = _asc_u32_inv(_order_stats(_asc_u32(s.reshape(-1)), pos_asc))

    g = jnp.sum((s[:, :, None] < bvals[None, None, :]).astype(jnp.int32),
                axis=2)
    c_thr = jnp.float32(-1e-8)
    ge = jnp.sum((s[:, :, None] >= bvals[None, None, :]).astype(jnp.int32),
                 axis=1)
    gp = jnp.sum(((s[:, :, None] >= bvals[None, None, :])
                  & (s[:, :, None] > c_thr)).astype(jnp.int32), axis=1)
    gn = jnp.sum(((s[:, :, None] >= bvals[None, None, :])
                  & (s[:, :, None] < c_thr)).astype(jnp.int32), axis=1)
    npos = jnp.sum((s > c_thr).astype(jnp.int32), axis=1, keepdims=True)
    nneg = jnp.sum((s < c_thr).astype(jnp.int32), axis=1, keepdims=True)
    zero_col = jnp.zeros((B, 1), jnp.int32)
    ge_full = jnp.concatenate([zero_col, ge, jnp.full((B, 1), N, jnp.int32)], axis=1)
    gp_full = jnp.concatenate([zero_col, gp, npos], axis=1)
    gn_full = jnp.concatenate([zero_col, gn, nneg], axis=1)
    counts = ge_full[:, 1:] - ge_full[:, :-1]

    bin_prob = jnp.broadcast_to(
        jax.nn.softmax(bin_prob_logits)[None, :], (B, NUM_BINS))
    k = _alloc_points(bin_prob, counts, STRIDE)
    start = jnp.concatenate(
        [jnp.zeros((B, 1), jnp.int32), jnp.cumsum(k, axis=1)[:, :-1]], axis=1)

    t = s + jnp.float32(1e-8)
    key = lax.bitcast_convert_type(~_asc_u32(t), jnp.int32)
    cls = jnp.where(t > 0, 0, jnp.where(t < 0, 2, 1)).astype(jnp.int32)
    val = (jnp.broadcast_to(jnp.arange(N, dtype=jnp.int32)[None, :], (B, N))
           | (g << 16) | (cls << 20))

    n1 = gp_full[:, 1:] - gp_full[:, :-1]
    n4 = gn_full[:, 1:] - gn_full[:, :-1]
    a1 = jnp.minimum(k, n1)
    a2 = jnp.minimum(k - a1, N - n1 - n4)
    a3 = k - a1 - a2

    def pad16(x):
        return jnp.pad(x, ((0, 0), (0, L - NUM_BINS)))

    prm = jnp.stack(
        [pad16(start), pad16(a1), pad16(a2), pad16(a3),
         pad16(start + a1), pad16(start + a1 + a2),
         jnp.zeros((B, L), jnp.int32), jnp.zeros((B, L), jnp.int32)],
        axis=1).reshape(B * 8 * L)

    grouper = _make_sc_kernel(B, N, D, M)
    new_points, idx = grouper(key.reshape(-1), val.reshape(-1), prm,
                              points.reshape(B * N // 2, 2 * D))
    return new_points.reshape(B, M, D), idx.reshape(B, 1, M)


# speedup vs baseline: 2.3661x; 1.1931x over previous
import functools

import jax
import jax.numpy as jnp
from jax import lax
from jax.experimental import pallas as pl
from jax.experimental.pallas import tpu as pltpu
from jax.experimental.pallas import tpu_sc as plsc

NUM_BINS = 6
STRIDE = 4
L = 16


def _alloc_points(bin_prob, max_num_points, stride):
    total = jnp.sum(max_num_points[0, :]) // stride
    B, num_bins = bin_prob.shape
    p = bin_prob * max_num_points.astype(bin_prob.dtype) + 1e-10
    chosen = jnp.zeros_like(p)
    mnp_f = max_num_points.astype(p.dtype)
    for _ in range(num_bins):
        p = p / jnp.sum(p, axis=1, keepdims=True)
        num_to_choose = total.astype(p.dtype) - jnp.sum(chosen, axis=1, keepdims=True)
        chosen = chosen + p * num_to_choose
        chosen = jnp.where(chosen >= mnp_f, mnp_f, chosen)
        p = p * jnp.where(chosen >= mnp_f, 0.0, 1.0)
    chosen = chosen.astype(jnp.int32)
    adj = jnp.argmax(mnp_f - chosen.astype(p.dtype), axis=1)
    deficit = total.astype(jnp.int32) - jnp.sum(chosen, axis=1)
    chosen = chosen.at[jnp.arange(B), adj].add(deficit)
    return chosen


def _asc_u32(x):
    ub = lax.bitcast_convert_type(x, jnp.uint32)
    return jnp.where(ub >= jnp.uint32(0x80000000), ~ub, ub | jnp.uint32(0x80000000))


def _asc_u32_inv(r):
    ub = jnp.where(r >= jnp.uint32(0x80000000), r ^ jnp.uint32(0x80000000), ~r)
    return lax.bitcast_convert_type(ub, jnp.float32)


def _order_stats(u_flat, pos):
    nt = pos.shape[0]

    def body(i, r):
        sh = (jnp.uint32(30) - 2 * i.astype(jnp.uint32))
        cands = (r[:, None]
                 | (jnp.uint32(1) + jnp.arange(3, dtype=jnp.uint32))[None, :] << sh)
        cnt = jnp.sum((cands.reshape(-1)[:, None] > u_flat[None, :]).astype(jnp.int32),
                      axis=1).reshape(nt, 3)
        d = jnp.sum((cnt <= pos[:, None]).astype(jnp.uint32), axis=1)
        return r | (d << sh)

    return lax.fori_loop(0, 16, body, jnp.zeros(pos.shape, jnp.uint32))


def _make_sc_kernel(B, N, D, M):
    CH = N // L
    mesh = plsc.VectorSubcoreMesh(core_axis_name="c", subcore_axis_name="s")
    NC = 2

    @functools.partial(
        pl.kernel,
        out_type=(
            jax.ShapeDtypeStruct((B * M,), jnp.int32),
            jax.ShapeDtypeStruct((B * M,), jnp.int32),
            jax.ShapeDtypeStruct((B * M,), jnp.int32),
        ),
        mesh=mesh,
        scratch_types=[
            pltpu.VMEM((N,), jnp.int32),
            pltpu.VMEM((N,), jnp.int32),
            pltpu.VMEM((N,), jnp.int32),
            pltpu.VMEM((N,), jnp.int32),
            pltpu.VMEM((N,), jnp.int32),
            pltpu.VMEM((8 * L,), jnp.int32),
            pltpu.VMEM((M,), jnp.int32),
            pltpu.VMEM((M,), jnp.int32),
            pltpu.VMEM((M,), jnp.int32),
            pltpu.VMEM((L,), jnp.int32),
            pltpu.VMEM((L,), jnp.int32),
        ],
        compiler_params=pltpu.CompilerParams(needs_layout_passes=False),
    )
    def sorter(key_h, val_h, prm_h, idxo_h, adj_h, par_h,
               keyA, valA, keyB, valB, chunk, prmv, outidx, idxadj, parbuf,
               c1, c3):
        cid = lax.axis_index("c")
        sid = lax.axis_index("s")
        b = sid * NC + cid

        pltpu.sync_copy(key_h.at[pl.ds(b * N, N)], keyA)
        pltpu.sync_copy(val_h.at[pl.ds(b * N, N)], valA)
        pltpu.sync_copy(prm_h.at[pl.ds(b * 8 * L, 8 * L)], prmv)

        zvec = jnp.zeros((L,), jnp.int32)

        bufs = [(keyA, valA, keyB, valB), (keyB, valB, keyA, valA)]
        for p in range(8):
            Kin, Vin, Kout, Vout = bufs[p % 2]
            sh = jnp.int32(4 * p)

            @plsc.parallel_loop(0, CH, unroll=4)
            def hist_body(c, Kin=Kin, sh=sh):
                kv = Kin[pl.ds(c * L, L)]
                d = lax.shift_right_logical(kv, sh) & jnp.int32(15)
                cnt, last = plsc.scan_count(d)
                chunk[pl.ds(c * L, L)] = zvec
                plsc.store_scatter(chunk, [c * L + d], cnt, mask=last)

            def prefix_body(c, acc):
                row = chunk[pl.ds(c * L, L)]
                chunk[pl.ds(c * L, L)] = acc
                return acc + row

            tot = lax.fori_loop(0, CH, prefix_body, zvec)
            gbase = plsc.cumsum(tot) - tot

            @plsc.parallel_loop(0, CH, unroll=4)
            def perm_body(c, Kin=Kin, Vin=Vin, Kout=Kout, Vout=Vout,
                          sh=sh, gbase=gbase):
                kv = Kin[pl.ds(c * L, L)]
                vv = Vin[pl.ds(c * L, L)]
                d = lax.shift_right_logical(kv, sh) & jnp.int32(15)
                cnt, _last = plsc.scan_count(d)
                basev = gbase + chunk[pl.ds(c * L, L)]
                rank = jnp.take(basev, d, axis=0) + cnt - 1
                plsc.store_scatter(Kout, [rank], kv)
                plsc.store_scatter(Vout, [rank], vv)

        startv = prmv[pl.ds(0 * L, L)]
        a1v = prmv[pl.ds(1 * L, L)]
        a2v = prmv[pl.ds(2 * L, L)]
        a3v = prmv[pl.ds(3 * L, L)]
        s2v = prmv[pl.ds(4 * L, L)]
        s3v = prmv[pl.ds(5 * L, L)]
        c1[...] = zvec
        c3[...] = zvec

        def selA(c, _):
            vv = valA[pl.ds(c * L, L)]
            gv = (vv >> 16) & jnp.int32(7)
            cls = (vv >> 20) & jnp.int32(3)
            iv = vv & jnp.int32(0xFFFF)
            m1 = cls == 0
            cnt1, last1 = plsc.scan_count(gv, m1)
            r1 = plsc.load_gather(c1, [gv]) + cnt1 - 1
            sel1 = m1 & (r1 < jnp.take(a1v, gv, axis=0))
            plsc.store_scatter(
                outidx, [jnp.take(startv, gv, axis=0) + r1], iv, mask=sel1)
            plsc.addupdate_scatter(c1, [gv], cnt1, mask=last1)
            m3 = cls == 2
            cnt3, last3 = plsc.scan_count(gv, m3)
            r3 = plsc.load_gather(c3, [gv]) + cnt3 - 1
            sel3 = m3 & (r3 < jnp.take(a3v, gv, axis=0))
            plsc.store_scatter(
                outidx, [jnp.take(s3v, gv, axis=0) + r3], iv, mask=sel3)
            plsc.addupdate_scatter(c3, [gv], cnt3, mask=last3)
            return 0

        lax.fori_loop(0, N // L, selA, 0)

        pltpu.sync_copy(val_h.at[pl.ds(b * N, N)], valB)
        a2b = [jnp.take(a2v, jnp.full((L,), j, jnp.int32), axis=0)
               for j in range(NUM_BINS)]
        s2b = [jnp.take(s2v, jnp.full((L,), j, jnp.int32), axis=0)
               for j in range(NUM_BINS)]
        lane15 = jnp.full((L,), L - 1, jnp.int32)

        def selB(c, bases):
            vv = valB[pl.ds(c * L, L)]
            gv = (vv >> 16) & jnp.int32(7)
            cls = (vv >> 20) & jnp.int32(3)
            iv = vv & jnp.int32(0xFFFF)
            new_bases = []
            for j in range(NUM_BINS):
                elig = (gv != j) | (cls == 1)
                ei = jnp.where(elig, 1, 0).astype(jnp.int32)
                incl = plsc.cumsum(ei)
                excl = bases[j] + incl - ei
                sel = elig & (excl < a2b[j])
                plsc.store_scatter(outidx, [s2b[j] + excl], iv, mask=sel)
                new_bases.append(bases[j] + jnp.take(incl, lane15, axis=0))
            return tuple(new_bases)

        lax.fori_loop(0, N // L, selB, (zvec,) * NUM_BINS)

        pltpu.sync_copy(outidx, idxo_h.at[pl.ds(b * M, M)])

        off = b * (N // 2)

        @plsc.parallel_loop(0, M // L, unroll=4)
        def adj_body(c):
            ov = outidx[pl.ds(c * L, L)]
            idxadj[pl.ds(c * L, L)] = (
                lax.shift_right_logical(ov, jnp.int32(1)) + off)
            parbuf[pl.ds(c * L, L)] = ov & jnp.int32(1)

        pltpu.sync_copy(idxadj, adj_h.at[pl.ds(b * M, M)])
        pltpu.sync_copy(parbuf, par_h.at[pl.ds(b * M, M)])

    @functools.partial(
        pl.kernel,
        out_type=jax.ShapeDtypeStruct((B * M, D), jnp.float32),
        mesh=mesh,
        scratch_types=[
            pltpu.VMEM((M,), jnp.int32),
            pltpu.VMEM((M,), jnp.int32),
            pltpu.VMEM((M // 8, 2 * D), jnp.float32),
            pltpu.VMEM((M // 8, D), jnp.float32),
            pltpu.SemaphoreType.DMA,
        ],
        compiler_params=pltpu.CompilerParams(needs_layout_passes=False),
    )
    def gatherer(adj_h, par_h, pts_h, npo_h, idxadj, parbuf, gbuf, stage, sem):
        cid = lax.axis_index("c")
        sid = lax.axis_index("s")
        b = sid * NC + cid
        pltpu.sync_copy(adj_h.at[pl.ds(b * M, M)], idxadj)
        pltpu.sync_copy(par_h.at[pl.ds(b * M, M)], parbuf)

        iota = lax.broadcasted_iota(jnp.int32, (L,), 0)
        q = M // 8
        for chk in range(8):
            cp = pltpu.async_copy(
                pts_h.at[idxadj.at[pl.ds(chk * q, q)]], gbuf, sem)
            cp.wait()

            @plsc.parallel_loop(0, 4 * q, unroll=4)
            def fill_body(i, chk=chk):
                r = lax.shift_right_logical(i, jnp.int32(2))
                qp = i & jnp.int32(3)
                rv = jnp.full((L,), r, jnp.int32)
                par = plsc.load_gather(parbuf, [chk * q + rv])
                dcol = qp * L + iota
                vals = plsc.load_gather(gbuf, [rv, par * D + dcol])
                plsc.store_scatter(stage, [rv, dcol], vals)

            pltpu.sync_copy(stage, npo_h.at[pl.ds(b * M + chk * q, q)])

    return sorter, gatherer


def kernel(points, attention_point_score, bin_prob_logits):
    B, N, D = points.shape
    M = N // STRIDE
    score = attention_point_score

    m = jnp.mean(score, axis=2, keepdims=True)
    sd = jnp.std(score, axis=2, keepdims=True)
    s = ((score - m) / sd)[:, 0, :]

    n = B * N
    idxq = (jnp.arange(1, NUM_BINS) / NUM_BINS * n).astype(jnp.int32)
    pos_asc = jnp.int32(n - 1) - idxq
    bvals = _asc_u32_inv(_order_stats(_asc_u32(s.reshape(-1)), pos_asc))

    g = jnp.sum((s[:, :, None] < bvals[None, None, :]).astype(jnp.int32),
                axis=2)
    c_thr = jnp.float32(-1e-8)
    ge = jnp.sum((s[:, :, None] >= bvals[None, None, :]).astype(jnp.int32),
                 axis=1)
    gp = jnp.sum(((s[:, :, None] >= bvals[None, None, :])
                  & (s[:, :, None] > c_thr)).astype(jnp.int32), axis=1)
    gn = jnp.sum(((s[:, :, None] >= bvals[None, None, :])
                  & (s[:, :, None] < c_thr)).astype(jnp.int32), axis=1)
    npos = jnp.sum((s > c_thr).astype(jnp.int32), axis=1, keepdims=True)
    nneg = jnp.sum((s < c_thr).astype(jnp.int32), axis=1, keepdims=True)
    zero_col = jnp.zeros((B, 1), jnp.int32)
    ge_full = jnp.concatenate([zero_col, ge, jnp.full((B, 1), N, jnp.int32)], axis=1)
    gp_full = jnp.concatenate([zero_col, gp, npos], axis=1)
    gn_full = jnp.concatenate([zero_col, gn, nneg], axis=1)
    counts = ge_full[:, 1:] - ge_full[:, :-1]

    bin_prob = jnp.broadcast_to(
        jax.nn.softmax(bin_prob_logits)[None, :], (B, NUM_BINS))
    k = _alloc_points(bin_prob, counts, STRIDE)
    start = jnp.concatenate(
        [jnp.zeros((B, 1), jnp.int32), jnp.cumsum(k, axis=1)[:, :-1]], axis=1)

    t = s + jnp.float32(1e-8)
    key = lax.bitcast_convert_type(~_asc_u32(t), jnp.int32)
    cls = jnp.where(t > 0, 0, jnp.where(t < 0, 2, 1)).astype(jnp.int32)
    val = (jnp.broadcast_to(jnp.arange(N, dtype=jnp.int32)[None, :], (B, N))
           | (g << 16) | (cls << 20))

    n1 = gp_full[:, 1:] - gp_full[:, :-1]
    n4 = gn_full[:, 1:] - gn_full[:, :-1]
    a1 = jnp.minimum(k, n1)
    a2 = jnp.minimum(k - a1, N - n1 - n4)
    a3 = k - a1 - a2

    def pad16(x):
        return jnp.pad(x, ((0, 0), (0, L - NUM_BINS)))

    prm = jnp.stack(
        [pad16(start), pad16(a1), pad16(a2), pad16(a3),
         pad16(start + a1), pad16(start + a1 + a2),
         jnp.zeros((B, L), jnp.int32), jnp.zeros((B, L), jnp.int32)],
        axis=1).reshape(B * 8 * L)

    sorter, gatherer = _make_sc_kernel(B, N, D, M)
    idx, adj, par = sorter(key.reshape(-1), val.reshape(-1), prm)
    new_points = gatherer(adj, par, points.reshape(B * N // 2, 2 * D))
    return new_points.reshape(B, M, D), idx.reshape(B, 1, M)


# speedup vs baseline: 2.3716x; 1.0024x over previous
import functools

import jax
import jax.numpy as jnp
from jax import lax
from jax.experimental import pallas as pl
from jax.experimental.pallas import tpu as pltpu
from jax.experimental.pallas import tpu_sc as plsc

NUM_BINS = 6
STRIDE = 4
L = 16


def _alloc_points(bin_prob, max_num_points, stride):
    total = jnp.sum(max_num_points[0, :]) // stride
    B, num_bins = bin_prob.shape
    p = bin_prob * max_num_points.astype(bin_prob.dtype) + 1e-10
    chosen = jnp.zeros_like(p)
    mnp_f = max_num_points.astype(p.dtype)
    for _ in range(num_bins):
        p = p / jnp.sum(p, axis=1, keepdims=True)
        num_to_choose = total.astype(p.dtype) - jnp.sum(chosen, axis=1, keepdims=True)
        chosen = chosen + p * num_to_choose
        chosen = jnp.where(chosen >= mnp_f, mnp_f, chosen)
        p = p * jnp.where(chosen >= mnp_f, 0.0, 1.0)
    chosen = chosen.astype(jnp.int32)
    adj = jnp.argmax(mnp_f - chosen.astype(p.dtype), axis=1)
    deficit = total.astype(jnp.int32) - jnp.sum(chosen, axis=1)
    chosen = chosen.at[jnp.arange(B), adj].add(deficit)
    return chosen


def _asc_u32(x):
    ub = lax.bitcast_convert_type(x, jnp.uint32)
    return jnp.where(ub >= jnp.uint32(0x80000000), ~ub, ub | jnp.uint32(0x80000000))


def _asc_u32_inv(r):
    ub = jnp.where(r >= jnp.uint32(0x80000000), r ^ jnp.uint32(0x80000000), ~r)
    return lax.bitcast_convert_type(ub, jnp.float32)


def _order_stats(u_flat, pos):
    nt = pos.shape[0]

    def body(i, r):
        sh = (jnp.uint32(30) - 2 * i.astype(jnp.uint32))
        cands = (r[:, None]
                 | (jnp.uint32(1) + jnp.arange(3, dtype=jnp.uint32))[None, :] << sh)
        cnt = jnp.sum((cands.reshape(-1)[:, None] > u_flat[None, :]).astype(jnp.int32),
                      axis=1).reshape(nt, 3)
        d = jnp.sum((cnt <= pos[:, None]).astype(jnp.uint32), axis=1)
        return r | (d << sh)

    return lax.fori_loop(0, 16, body, jnp.zeros(pos.shape, jnp.uint32))


def _make_sc_kernel(B, N, D, M):
    CH = N // L
    mesh = plsc.VectorSubcoreMesh(core_axis_name="c", subcore_axis_name="s")
    NC = 2

    @functools.partial(
        pl.kernel,
        out_type=(
            jax.ShapeDtypeStruct((B * M,), jnp.int32),
            jax.ShapeDtypeStruct((B * M,), jnp.int32),
            jax.ShapeDtypeStruct((B * M,), jnp.int32),
        ),
        mesh=mesh,
        scratch_types=[
            pltpu.VMEM((N,), jnp.int32),
            pltpu.VMEM((N,), jnp.int32),
            pltpu.VMEM((N,), jnp.int32),
            pltpu.VMEM((N,), jnp.int32),
            pltpu.VMEM((N,), jnp.int32),
            pltpu.VMEM((8 * L,), jnp.int32),
            pltpu.VMEM((M,), jnp.int32),
            pltpu.VMEM((M,), jnp.int32),
            pltpu.VMEM((M,), jnp.int32),
            pltpu.VMEM((L,), jnp.int32),
            pltpu.VMEM((L,), jnp.int32),
        ],
        compiler_params=pltpu.CompilerParams(needs_layout_passes=False, use_tc_tiling_on_sc=True),
    )
    def sorter(key_h, val_h, prm_h, idxo_h, adj_h, par_h,
               keyA, valA, keyB, valB, chunk, prmv, outidx, idxadj, parbuf,
               c1, c3):
        cid = lax.axis_index("c")
        sid = lax.axis_index("s")
        b = sid * NC + cid

        pltpu.sync_copy(key_h.at[pl.ds(b * N, N)], keyA)
        pltpu.sync_copy(val_h.at[pl.ds(b * N, N)], valA)
        pltpu.sync_copy(prm_h.at[pl.ds(b * 8 * L, 8 * L)], prmv)

        zvec = jnp.zeros((L,), jnp.int32)

        bufs = [(keyA, valA, keyB, valB), (keyB, valB, keyA, valA)]
        for p in range(8):
            Kin, Vin, Kout, Vout = bufs[p % 2]
            sh = jnp.int32(4 * p)

            @plsc.parallel_loop(0, CH, unroll=4)
            def hist_body(c, Kin=Kin, sh=sh):
                kv = Kin[pl.ds(c * L, L)]
                d = lax.shift_right_logical(kv, sh) & jnp.int32(15)
                cnt, last = plsc.scan_count(d)
                chunk[pl.ds(c * L, L)] = zvec
                plsc.store_scatter(chunk, [c * L + d], cnt, mask=last)

            def prefix_body(c, acc):
                row = chunk[pl.ds(c * L, L)]
                chunk[pl.ds(c * L, L)] = acc
                return acc + row

            tot = lax.fori_loop(0, CH, prefix_body, zvec)
            gbase = plsc.cumsum(tot) - tot

            @plsc.parallel_loop(0, CH, unroll=4)
            def perm_body(c, Kin=Kin, Vin=Vin, Kout=Kout, Vout=Vout,
                          sh=sh, gbase=gbase):
                kv = Kin[pl.ds(c * L, L)]
                vv = Vin[pl.ds(c * L, L)]
                d = lax.shift_right_logical(kv, sh) & jnp.int32(15)
                cnt, _last = plsc.scan_count(d)
                basev = gbase + chunk[pl.ds(c * L, L)]
                rank = jnp.take(basev, d, axis=0) + cnt - 1
                plsc.store_scatter(Kout, [rank], kv)
                plsc.store_scatter(Vout, [rank], vv)

        startv = prmv[pl.ds(0 * L, L)]
        a1v = prmv[pl.ds(1 * L, L)]
        a2v = prmv[pl.ds(2 * L, L)]
        a3v = prmv[pl.ds(3 * L, L)]
        s2v = prmv[pl.ds(4 * L, L)]
        s3v = prmv[pl.ds(5 * L, L)]
        c1[...] = zvec
        c3[...] = zvec

        def selA(c, _):
            vv = valA[pl.ds(c * L, L)]
            gv = (vv >> 16) & jnp.int32(7)
            cls = (vv >> 20) & jnp.int32(3)
            iv = vv & jnp.int32(0xFFFF)
            m1 = cls == 0
            cnt1, last1 = plsc.scan_count(gv, m1)
            r1 = plsc.load_gather(c1, [gv]) + cnt1 - 1
            sel1 = m1 & (r1 < jnp.take(a1v, gv, axis=0))
            plsc.store_scatter(
                outidx, [jnp.take(startv, gv, axis=0) + r1], iv, mask=sel1)
            plsc.addupdate_scatter(c1, [gv], cnt1, mask=last1)
            m3 = cls == 2
            cnt3, last3 = plsc.scan_count(gv, m3)
            r3 = plsc.load_gather(c3, [gv]) + cnt3 - 1
            sel3 = m3 & (r3 < jnp.take(a3v, gv, axis=0))
            plsc.store_scatter(
                outidx, [jnp.take(s3v, gv, axis=0) + r3], iv, mask=sel3)
            plsc.addupdate_scatter(c3, [gv], cnt3, mask=last3)
            return 0

        lax.fori_loop(0, N // L, selA, 0)

        pltpu.sync_copy(val_h.at[pl.ds(b * N, N)], valB)
        a2b = [jnp.take(a2v, jnp.full((L,), j, jnp.int32), axis=0)
               for j in range(NUM_BINS)]
        s2b = [jnp.take(s2v, jnp.full((L,), j, jnp.int32), axis=0)
               for j in range(NUM_BINS)]
        lane15 = jnp.full((L,), L - 1, jnp.int32)

        def selB(c, bases):
            vv = valB[pl.ds(c * L, L)]
            gv = (vv >> 16) & jnp.int32(7)
            cls = (vv >> 20) & jnp.int32(3)
            iv = vv & jnp.int32(0xFFFF)
            new_bases = []
            for j in range(NUM_BINS):
                elig = (gv != j) | (cls == 1)
                ei = jnp.where(elig, 1, 0).astype(jnp.int32)
                incl = plsc.cumsum(ei)
                excl = bases[j] + incl - ei
                sel = elig & (excl < a2b[j])
                plsc.store_scatter(outidx, [s2b[j] + excl], iv, mask=sel)
                new_bases.append(bases[j] + jnp.take(incl, lane15, axis=0))
            return tuple(new_bases)

        lax.fori_loop(0, N // L, selB, (zvec,) * NUM_BINS)

        pltpu.sync_copy(outidx, idxo_h.at[pl.ds(b * M, M)])

        off = b * (N // 2)

        @plsc.parallel_loop(0, M // L, unroll=4)
        def adj_body(c):
            ov = outidx[pl.ds(c * L, L)]
            idxadj[pl.ds(c * L, L)] = (
                lax.shift_right_logical(ov, jnp.int32(1)) + off)
            parbuf[pl.ds(c * L, L)] = ov & jnp.int32(1)

        pltpu.sync_copy(idxadj, adj_h.at[pl.ds(b * M, M)])
        pltpu.sync_copy(parbuf, par_h.at[pl.ds(b * M, M)])

    @functools.partial(
        pl.kernel,
        out_type=jax.ShapeDtypeStruct((B * M, D), jnp.float32),
        mesh=mesh,
        scratch_types=[
            pltpu.VMEM((M,), jnp.int32),
            pltpu.VMEM((M,), jnp.int32),
            pltpu.VMEM((M // 8, 2 * D), jnp.float32),
            pltpu.VMEM((M // 8, D), jnp.float32),
            pltpu.SemaphoreType.DMA,
        ],
        compiler_params=pltpu.CompilerParams(needs_layout_passes=False, use_tc_tiling_on_sc=True),
    )
    def gatherer(adj_h, par_h, pts_h, npo_h, idxadj, parbuf, gbuf, stage, sem):
        cid = lax.axis_index("c")
        sid = lax.axis_index("s")
        b = sid * NC + cid
        pltpu.sync_copy(adj_h.at[pl.ds(b * M, M)], idxadj)
        pltpu.sync_copy(par_h.at[pl.ds(b * M, M)], parbuf)

        iota = lax.broadcasted_iota(jnp.int32, (L,), 0)
        q = M // 8
        for chk in range(8):
            cp = pltpu.async_copy(
                pts_h.at[idxadj.at[pl.ds(chk * q, q)]], gbuf, sem)
            cp.wait()

            @plsc.parallel_loop(0, 4 * q, unroll=4)
            def fill_body(i, chk=chk):
                r = lax.shift_right_logical(i, jnp.int32(2))
                qp = i & jnp.int32(3)
                rv = jnp.full((L,), r, jnp.int32)
                par = plsc.load_gather(parbuf, [chk * q + rv])
                dcol = qp * L + iota
                vals = plsc.load_gather(gbuf, [rv, par * D + dcol])
                plsc.store_scatter(stage, [rv, dcol], vals)

            pltpu.sync_copy(stage, npo_h.at[pl.ds(b * M + chk * q, q)])

    return sorter, gatherer


def kernel(points, attention_point_score, bin_prob_logits):
    B, N, D = points.shape
    M = N // STRIDE
    score = attention_point_score

    m = jnp.mean(score, axis=2, keepdims=True)
    sd = jnp.std(score, axis=2, keepdims=True)
    s = ((score - m) / sd)[:, 0, :]

    n = B * N
    idxq = (jnp.arange(1, NUM_BINS) / NUM_BINS * n).astype(jnp.int32)
    pos_asc = jnp.int32(n - 1) - idxq
    bvals = _asc_u32_inv(_order_stats(_asc_u32(s.reshape(-1)), pos_asc))

    g = jnp.sum((s[:, :, None] < bvals[None, None, :]).astype(jnp.int32),
                axis=2)
    c_thr = jnp.float32(-1e-8)
    ge = jnp.sum((s[:, :, None] >= bvals[None, None, :]).astype(jnp.int32),
                 axis=1)
    gp = jnp.sum(((s[:, :, None] >= bvals[None, None, :])
                  & (s[:, :, None] > c_thr)).astype(jnp.int32), axis=1)
    gn = jnp.sum(((s[:, :, None] >= bvals[None, None, :])
                  & (s[:, :, None] < c_thr)).astype(jnp.int32), axis=1)
    npos = jnp.sum((s > c_thr).astype(jnp.int32), axis=1, keepdims=True)
    nneg = jnp.sum((s < c_thr).astype(jnp.int32), axis=1, keepdims=True)
    zero_col = jnp.zeros((B, 1), jnp.int32)
    ge_full = jnp.concatenate([zero_col, ge, jnp.full((B, 1), N, jnp.int32)], axis=1)
    gp_full = jnp.concatenate([zero_col, gp, npos], axis=1)
    gn_full = jnp.concatenate([zero_col, gn, nneg], axis=1)
    counts = ge_full[:, 1:] - ge_full[:, :-1]

    bin_prob = jnp.broadcast_to(
        jax.nn.softmax(bin_prob_logits)[None, :], (B, NUM_BINS))
    k = _alloc_points(bin_prob, counts, STRIDE)
    start = jnp.concatenate(
        [jnp.zeros((B, 1), jnp.int32), jnp.cumsum(k, axis=1)[:, :-1]], axis=1)

    t = s + jnp.float32(1e-8)
    key = lax.bitcast_convert_type(~_asc_u32(t), jnp.int32)
    cls = jnp.where(t > 0, 0, jnp.where(t < 0, 2, 1)).astype(jnp.int32)
    val = (jnp.broadcast_to(jnp.arange(N, dtype=jnp.int32)[None, :], (B, N))
           | (g << 16) | (cls << 20))

    n1 = gp_full[:, 1:] - gp_full[:, :-1]
    n4 = gn_full[:, 1:] - gn_full[:, :-1]
    a1 = jnp.minimum(k, n1)
    a2 = jnp.minimum(k - a1, N - n1 - n4)
    a3 = k - a1 - a2

    def pad16(x):
        return jnp.pad(x, ((0, 0), (0, L - NUM_BINS)))

    prm = jnp.stack(
        [pad16(start), pad16(a1), pad16(a2), pad16(a3),
         pad16(start + a1), pad16(start + a1 + a2),
         jnp.zeros((B, L), jnp.int32), jnp.zeros((B, L), jnp.int32)],
        axis=1).reshape(B * 8 * L)

    sorter, gatherer = _make_sc_kernel(B, N, D, M)
    idx, adj, par = sorter(key.reshape(-1), val.reshape(-1), prm)
    new_points = gatherer(adj, par, points.reshape(B * N // 2, 2 * D))
    return new_points.reshape(B, M, D), idx.reshape(B, 1, M)


# speedup vs baseline: 2.3719x; 1.0001x over previous
import functools

import jax
import jax.numpy as jnp
from jax import lax
from jax.experimental import pallas as pl
from jax.experimental.pallas import tpu as pltpu
from jax.experimental.pallas import tpu_sc as plsc

NUM_BINS = 6
STRIDE = 4
L = 16


def _alloc_points(bin_prob, max_num_points, stride):
    total = jnp.sum(max_num_points[0, :]) // stride
    B, num_bins = bin_prob.shape
    p = bin_prob * max_num_points.astype(bin_prob.dtype) + 1e-10
    chosen = jnp.zeros_like(p)
    mnp_f = max_num_points.astype(p.dtype)
    for _ in range(num_bins):
        p = p / jnp.sum(p, axis=1, keepdims=True)
        num_to_choose = total.astype(p.dtype) - jnp.sum(chosen, axis=1, keepdims=True)
        chosen = chosen + p * num_to_choose
        chosen = jnp.where(chosen >= mnp_f, mnp_f, chosen)
        p = p * jnp.where(chosen >= mnp_f, 0.0, 1.0)
    chosen = chosen.astype(jnp.int32)
    adj = jnp.argmax(mnp_f - chosen.astype(p.dtype), axis=1)
    deficit = total.astype(jnp.int32) - jnp.sum(chosen, axis=1)
    chosen = chosen.at[jnp.arange(B), adj].add(deficit)
    return chosen


def _asc_u32(x):
    ub = lax.bitcast_convert_type(x, jnp.uint32)
    return jnp.where(ub >= jnp.uint32(0x80000000), ~ub, ub | jnp.uint32(0x80000000))


def _asc_u32_inv(r):
    ub = jnp.where(r >= jnp.uint32(0x80000000), r ^ jnp.uint32(0x80000000), ~r)
    return lax.bitcast_convert_type(ub, jnp.float32)


def _order_stats(u_flat, pos):
    nt = pos.shape[0]

    def body(i, r):
        sh = (jnp.uint32(30) - 2 * i.astype(jnp.uint32))
        cands = (r[:, None]
                 | (jnp.uint32(1) + jnp.arange(3, dtype=jnp.uint32))[None, :] << sh)
        cnt = jnp.sum((cands.reshape(-1)[:, None] > u_flat[None, :]).astype(jnp.int32),
                      axis=1).reshape(nt, 3)
        d = jnp.sum((cnt <= pos[:, None]).astype(jnp.uint32), axis=1)
        return r | (d << sh)

    return lax.fori_loop(0, 16, body, jnp.zeros(pos.shape, jnp.uint32))


def _make_sc_kernel(B, N, D, M):
    CH = N // L
    mesh = plsc.VectorSubcoreMesh(core_axis_name="c", subcore_axis_name="s")
    NC = 2

    @functools.partial(
        pl.kernel,
        out_type=(
            jax.ShapeDtypeStruct((B * M,), jnp.int32),
            jax.ShapeDtypeStruct((B * M,), jnp.int32),
            jax.ShapeDtypeStruct((B * M,), jnp.int32),
        ),
        mesh=mesh,
        scratch_types=[
            pltpu.VMEM((N,), jnp.int32),
            pltpu.VMEM((N,), jnp.int32),
            pltpu.VMEM((N,), jnp.int32),
            pltpu.VMEM((N,), jnp.int32),
            pltpu.VMEM((N,), jnp.int32),
            pltpu.VMEM((8 * L,), jnp.int32),
            pltpu.VMEM((M,), jnp.int32),
            pltpu.VMEM((M,), jnp.int32),
            pltpu.VMEM((M,), jnp.int32),
            pltpu.VMEM((L,), jnp.int32),
            pltpu.VMEM((L,), jnp.int32),
        ],
        compiler_params=pltpu.CompilerParams(needs_layout_passes=False),
    )
    def sorter(key_h, val_h, prm_h, idxo_h, adj_h, par_h,
               keyA, valA, keyB, valB, chunk, prmv, outidx, idxadj, parbuf,
               c1, c3):
        cid = lax.axis_index("c")
        sid = lax.axis_index("s")
        b = sid * NC + cid

        pltpu.sync_copy(key_h.at[pl.ds(b * N, N)], keyA)
        pltpu.sync_copy(val_h.at[pl.ds(b * N, N)], valA)
        pltpu.sync_copy(prm_h.at[pl.ds(b * 8 * L, 8 * L)], prmv)

        zvec = jnp.zeros((L,), jnp.int32)

        bufs = [(keyA, valA, keyB, valB), (keyB, valB, keyA, valA)]
        for p in range(8):
            Kin, Vin, Kout, Vout = bufs[p % 2]
            sh = jnp.int32(4 * p)

            @plsc.parallel_loop(0, CH, unroll=4)
            def hist_body(c, Kin=Kin, sh=sh):
                kv = Kin[pl.ds(c * L, L)]
                d = lax.shift_right_logical(kv, sh) & jnp.int32(15)
                cnt, last = plsc.scan_count(d)
                chunk[pl.ds(c * L, L)] = zvec
                plsc.store_scatter(chunk, [c * L + d], cnt, mask=last)

            def prefix_body(c, acc):
                row = chunk[pl.ds(c * L, L)]
                chunk[pl.ds(c * L, L)] = acc
                return acc + row

            tot = lax.fori_loop(0, CH, prefix_body, zvec)
            gbase = plsc.cumsum(tot) - tot

            @plsc.parallel_loop(0, CH, unroll=4)
            def perm_body(c, Kin=Kin, Vin=Vin, Kout=Kout, Vout=Vout,
                          sh=sh, gbase=gbase):
                kv = Kin[pl.ds(c * L, L)]
                vv = Vin[pl.ds(c * L, L)]
                d = lax.shift_right_logical(kv, sh) & jnp.int32(15)
                cnt, _last = plsc.scan_count(d)
                basev = gbase + chunk[pl.ds(c * L, L)]
                rank = jnp.take(basev, d, axis=0) + cnt - 1
                plsc.store_scatter(Kout, [rank], kv)
                plsc.store_scatter(Vout, [rank], vv)

        startv = prmv[pl.ds(0 * L, L)]
        a1v = prmv[pl.ds(1 * L, L)]
        a2v = prmv[pl.ds(2 * L, L)]
        a3v = prmv[pl.ds(3 * L, L)]
        s2v = prmv[pl.ds(4 * L, L)]
        s3v = prmv[pl.ds(5 * L, L)]
        c1[...] = zvec
        c3[...] = zvec

        def selA(c, _):
            vv = valA[pl.ds(c * L, L)]
            gv = (vv >> 16) & jnp.int32(7)
            cls = (vv >> 20) & jnp.int32(3)
            iv = vv & jnp.int32(0xFFFF)
            m1 = cls == 0
            cnt1, last1 = plsc.scan_count(gv, m1)
            r1 = plsc.load_gather(c1, [gv]) + cnt1 - 1
            sel1 = m1 & (r1 < jnp.take(a1v, gv, axis=0))
            plsc.store_scatter(
                outidx, [jnp.take(startv, gv, axis=0) + r1], iv, mask=sel1)
            plsc.addupdate_scatter(c1, [gv], cnt1, mask=last1)
            m3 = cls == 2
            cnt3, last3 = plsc.scan_count(gv, m3)
            r3 = plsc.load_gather(c3, [gv]) + cnt3 - 1
            sel3 = m3 & (r3 < jnp.take(a3v, gv, axis=0))
            plsc.store_scatter(
                outidx, [jnp.take(s3v, gv, axis=0) + r3], iv, mask=sel3)
            plsc.addupdate_scatter(c3, [gv], cnt3, mask=last3)
            return 0

        lax.fori_loop(0, N // L, selA, 0)

        pltpu.sync_copy(val_h.at[pl.ds(b * N, N)], valB)
        a2b = [jnp.take(a2v, jnp.full((L,), j, jnp.int32), axis=0)
               for j in range(NUM_BINS)]
        s2b = [jnp.take(s2v, jnp.full((L,), j, jnp.int32), axis=0)
               for j in range(NUM_BINS)]
        lane15 = jnp.full((L,), L - 1, jnp.int32)

        def selB(c, bases):
            vv = valB[pl.ds(c * L, L)]
            gv = (vv >> 16) & jnp.int32(7)
            cls = (vv >> 20) & jnp.int32(3)
            iv = vv & jnp.int32(0xFFFF)
            new_bases = []
            for j in range(NUM_BINS):
                elig = (gv != j) | (cls == 1)
                ei = jnp.where(elig, 1, 0).astype(jnp.int32)
                incl = plsc.cumsum(ei)
                excl = bases[j] + incl - ei
                sel = elig & (excl < a2b[j])
                plsc.store_scatter(outidx, [s2b[j] + excl], iv, mask=sel)
                new_bases.append(bases[j] + jnp.take(incl, lane15, axis=0))
            return tuple(new_bases)

        lax.fori_loop(0, N // L, selB, (zvec,) * NUM_BINS)

        pltpu.sync_copy(outidx, idxo_h.at[pl.ds(b * M, M)])

        off = b * (N // 2)

        @plsc.parallel_loop(0, M // L, unroll=4)
        def adj_body(c):
            ov = outidx[pl.ds(c * L, L)]
            idxadj[pl.ds(c * L, L)] = (
                lax.shift_right_logical(ov, jnp.int32(1)) + off)
            parbuf[pl.ds(c * L, L)] = ov & jnp.int32(1)

        pltpu.sync_copy(idxadj, adj_h.at[pl.ds(b * M, M)])
        pltpu.sync_copy(parbuf, par_h.at[pl.ds(b * M, M)])

    @functools.partial(
        pl.kernel,
        out_type=jax.ShapeDtypeStruct((B * M, D), jnp.float32),
        mesh=mesh,
        scratch_types=[
            pltpu.VMEM((M,), jnp.int32),
            pltpu.VMEM((M,), jnp.int32),
            pltpu.VMEM((M // 8, 2 * D), jnp.float32),
            pltpu.VMEM((M // 8, D), jnp.float32),
            pltpu.SemaphoreType.DMA,
        ],
        compiler_params=pltpu.CompilerParams(needs_layout_passes=False),
    )
    def gatherer(adj_h, par_h, pts_h, npo_h, idxadj, parbuf, gbuf, stage, sem):
        cid = lax.axis_index("c")
        sid = lax.axis_index("s")
        b = sid * NC + cid
        pltpu.sync_copy(adj_h.at[pl.ds(b * M, M)], idxadj)
        pltpu.sync_copy(par_h.at[pl.ds(b * M, M)], parbuf)

        iota = lax.broadcasted_iota(jnp.int32, (L,), 0)
        q = M // 8
        for chk in range(8):
            cp = pltpu.async_copy(
                pts_h.at[idxadj.at[pl.ds(chk * q, q)]], gbuf, sem)
            cp.wait()

            @plsc.parallel_loop(0, 4 * q, unroll=4)
            def fill_body(i, chk=chk):
                r = lax.shift_right_logical(i, jnp.int32(2))
                qp = i & jnp.int32(3)
                rv = jnp.full((L,), r, jnp.int32)
                par = plsc.load_gather(parbuf, [chk * q + rv])
                dcol = qp * L + iota
                vals = plsc.load_gather(gbuf, [rv, par * D + dcol])
                plsc.store_scatter(stage, [rv, dcol], vals)

            pltpu.sync_copy(stage, npo_h.at[pl.ds(b * M + chk * q, q)])

    return sorter, gatherer


def kernel(points, attention_point_score, bin_prob_logits):
    B, N, D = points.shape
    M = N // STRIDE
    score = attention_point_score

    m = jnp.mean(score, axis=2, keepdims=True)
    sd = jnp.std(score, axis=2, keepdims=True)
    s = ((score - m) / sd)[:, 0, :]

    n = B * N
    idxq = (jnp.arange(1, NUM_BINS) / NUM_BINS * n).astype(jnp.int32)
    pos_asc = jnp.int32(n - 1) - idxq
    bvals = _asc_u32_inv(_order_stats(_asc_u32(s.reshape(-1)), pos_asc))

    g = jnp.sum((s[:, :, None] < bvals[None, None, :]).astype(jnp.int32),
                axis=2)
    c_thr = jnp.float32(-1e-8)
    ge = jnp.sum((s[:, :, None] >= bvals[None, None, :]).astype(jnp.int32),
                 axis=1)
    gp = jnp.sum(((s[:, :, None] >= bvals[None, None, :])
                  & (s[:, :, None] > c_thr)).astype(jnp.int32), axis=1)
    gn = jnp.sum(((s[:, :, None] >= bvals[None, None, :])
                  & (s[:, :, None] < c_thr)).astype(jnp.int32), axis=1)
    npos = jnp.sum((s > c_thr).astype(jnp.int32), axis=1, keepdims=True)
    nneg = jnp.sum((s < c_thr).astype(jnp.int32), axis=1, keepdims=True)
    zero_col = jnp.zeros((B, 1), jnp.int32)
    ge_full = jnp.concatenate([zero_col, ge, jnp.full((B, 1), N, jnp.int32)], axis=1)
    gp_full = jnp.concatenate([zero_col, gp, npos], axis=1)
    gn_full = jnp.concatenate([zero_col, gn, nneg], axis=1)
    counts = ge_full[:, 1:] - ge_full[:, :-1]

    bin_prob = jnp.broadcast_to(
        jax.nn.softmax(bin_prob_logits)[None, :], (B, NUM_BINS))
    k = _alloc_points(bin_prob, counts, STRIDE)
    start = jnp.concatenate(
        [jnp.zeros((B, 1), jnp.int32), jnp.cumsum(k, axis=1)[:, :-1]], axis=1)

    t = s + jnp.float32(1e-8)
    key = lax.bitcast_convert_type(~_asc_u32(t), jnp.int32)
    cls = jnp.where(t > 0, 0, jnp.where(t < 0, 2, 1)).astype(jnp.int32)
    val = (jnp.broadcast_to(jnp.arange(N, dtype=jnp.int32)[None, :], (B, N))
           | (g << 16) | (cls << 20))

    n1 = gp_full[:, 1:] - gp_full[:, :-1]
    n4 = gn_full[:, 1:] - gn_full[:, :-1]
    a1 = jnp.minimum(k, n1)
    a2 = jnp.minimum(k - a1, N - n1 - n4)
    a3 = k - a1 - a2

    def pad16(x):
        return jnp.pad(x, ((0, 0), (0, L - NUM_BINS)))

    prm = jnp.stack(
        [pad16(start), pad16(a1), pad16(a2), pad16(a3),
         pad16(start + a1), pad16(start + a1 + a2),
         jnp.zeros((B, L), jnp.int32), jnp.zeros((B, L), jnp.int32)],
        axis=1).reshape(B * 8 * L)

    sorter, gatherer = _make_sc_kernel(B, N, D, M)
    idx, adj, par = sorter(key.reshape(-1), val.reshape(-1), prm)
    new_points = gatherer(adj, par, points.reshape(B * N // 2, 2 * D))
    return new_points.reshape(B, M, D), idx.reshape(B, 1, M)
